# baseline (device time: 62948 ns/iter reference)
import jax
import jax.numpy as jnp
from jax import lax
from jax.experimental import pallas as pl
from jax.experimental.pallas import tpu as pltpu

N_DEV = 16
F8 = jnp.float8_e4m3fn


def kernel(x, w_mat, scale_x, scale_w):
    m_total, k_shard = x.shape
    k_total, n_total = w_mat.shape
    m_blk = m_total // N_DEV

    def body(x_ref, w_ref, sx_ref, sw_ref, out_ref,
             x8_ref, recv_ref, send_sems, recv_sems):
        c = pl.program_id(0)
        me = lax.axis_index("i")

        @pl.when(c == 0)
        def _setup():
            bsem = pltpu.get_barrier_semaphore()
            for d in range(N_DEV):
                pl.semaphore_signal(
                    bsem, inc=1,
                    device_id=(d,), device_id_type=pl.DeviceIdType.MESH,
                )
            pl.semaphore_wait(bsem, N_DEV)

            x8_ref[...] = x_ref[...].astype(F8)
            recv_ref[me] = x8_ref[pl.ds(me * m_blk, m_blk), :]

            for d in range(N_DEV):
                @pl.when(d != me)
                def _send():
                    rdma = pltpu.make_async_remote_copy(
                        src_ref=x8_ref.at[pl.ds(d * m_blk, m_blk), :],
                        dst_ref=recv_ref.at[me],
                        send_sem=send_sems.at[d],
                        recv_sem=recv_sems.at[me],
                        device_id=(d,),
                        device_id_type=pl.DeviceIdType.MESH,
                    )
                    rdma.start()

        @pl.when(c != me)
        def _wait():
            recv = pltpu.make_async_remote_copy(
                src_ref=x8_ref.at[pl.ds(0, m_blk), :],
                dst_ref=recv_ref.at[c],
                send_sem=send_sems.at[c],
                recv_sem=recv_sems.at[c],
                device_id=(me,),
                device_id_type=pl.DeviceIdType.MESH,
            )
            recv.wait_recv()

        a = recv_ref[c].astype(jnp.bfloat16)
        wb = w_ref[...].astype(jnp.bfloat16)
        part = jnp.dot(a, wb, preferred_element_type=jnp.float32)

        @pl.when(c == 0)
        def _init():
            out_ref[...] = part

        @pl.when(jnp.logical_and(c != 0, c != N_DEV - 1))
        def _accum():
            out_ref[...] += part

        @pl.when(c == N_DEV - 1)
        def _finish():
            s = sx_ref[0] * sw_ref[0]
            out_ref[...] = jnp.maximum((out_ref[...] + part) * s, 0.0)
            for d in range(N_DEV):
                @pl.when(d != me)
                def _drain():
                    sd = pltpu.make_async_remote_copy(
                        src_ref=x8_ref.at[pl.ds(d * m_blk, m_blk), :],
                        dst_ref=recv_ref.at[me],
                        send_sem=send_sems.at[d],
                        recv_sem=recv_sems.at[me],
                        device_id=(d,),
                        device_id_type=pl.DeviceIdType.MESH,
                    )
                    sd.wait_send()

    return pl.pallas_call(
        body,
        grid=(N_DEV,),
        out_shape=jax.ShapeDtypeStruct((m_blk, n_total), jnp.float32),
        in_specs=[
            pl.BlockSpec((m_total, k_shard), lambda c: (0, 0)),
            pl.BlockSpec((m_blk, n_total), lambda c: (c, 0)),
            pl.BlockSpec(memory_space=pltpu.SMEM),
            pl.BlockSpec(memory_space=pltpu.SMEM),
        ],
        out_specs=pl.BlockSpec((m_blk, n_total), lambda c: (0, 0)),
        scratch_shapes=[
            pltpu.VMEM((m_total, k_shard), F8),
            pltpu.VMEM((N_DEV, m_blk, k_shard), F8),
            pltpu.SemaphoreType.DMA((N_DEV,)),
            pltpu.SemaphoreType.DMA((N_DEV,)),
        ],
        compiler_params=pltpu.CompilerParams(
            dimension_semantics=("arbitrary",),
            collective_id=0,
            has_side_effects=True,
        ),
    )(x, w_mat, scale_x, scale_w)


# device time: 62155 ns/iter; 1.0128x vs baseline; 1.0128x over previous
import jax
import jax.numpy as jnp
from jax import lax
from jax.experimental import pallas as pl
from jax.experimental.pallas import tpu as pltpu

N_DEV = 16
F8 = jnp.float8_e4m3fn
R = 4
S = 4
H = 2


def kernel(x, w_mat, scale_x, scale_w):
    m_total, k_shard = x.shape
    k_total, n_total = w_mat.shape
    m_blk = m_total // N_DEV
    rows = m_blk // S
    n_h = n_total // H

    def body(x_ref, w_ref, sx_ref, sw_ref, out_ref,
             wbuf_ref, x8_ref, recv_ref, wsems, send_sems, recv_sems):
        c = pl.program_id(0)
        me = lax.axis_index("i")

        def fill_slot(slot, chunk):
            for i in range(S):
                pltpu.make_async_copy(
                    w_ref.at[pl.ds(chunk * m_blk + i * rows, rows), :],
                    wbuf_ref.at[slot, pl.ds(i * rows, rows), :],
                    wsems.at[slot, i],
                ).start()

        def wait_slot(slot, chunk):
            for i in range(S):
                pltpu.make_async_copy(
                    w_ref.at[pl.ds(chunk * m_blk + i * rows, rows), :],
                    wbuf_ref.at[slot, pl.ds(i * rows, rows), :],
                    wsems.at[slot, i],
                ).wait()

        @pl.when(c == 0)
        def _setup():
            for r in range(R):
                fill_slot(r, (me + r) % N_DEV)

            x8_ref[...] = x_ref[...].astype(F8)

            bsem = pltpu.get_barrier_semaphore()
            for d in range(N_DEV):
                pl.semaphore_signal(
                    bsem, inc=1,
                    device_id=(d,), device_id_type=pl.DeviceIdType.MESH,
                )
            pl.semaphore_wait(bsem, N_DEV)

            recv_ref[me] = x8_ref[pl.ds(me * m_blk, m_blk), :]

            for off in range(1, N_DEV):
                d = (me + off) % N_DEV
                rdma = pltpu.make_async_remote_copy(
                    src_ref=x8_ref.at[pl.ds(d * m_blk, m_blk), :],
                    dst_ref=recv_ref.at[me],
                    send_sem=send_sems.at[d],
                    recv_sem=recv_sems.at[me],
                    device_id=(d,),
                    device_id_type=pl.DeviceIdType.MESH,
                )
                rdma.start()

        j = (me + c) % N_DEV
        r = c % R

        @pl.when(c != 0)
        def _wait_recv():
            recv = pltpu.make_async_remote_copy(
                src_ref=x8_ref.at[pl.ds(0, m_blk), :],
                dst_ref=recv_ref.at[j],
                send_sem=send_sems.at[j],
                recv_sem=recv_sems.at[j],
                device_id=(me,),
                device_id_type=pl.DeviceIdType.MESH,
            )
            recv.wait_recv()

        wait_slot(r, j)

        a = recv_ref[j].astype(jnp.bfloat16)
        for h in range(H):
            wb = wbuf_ref[r, :, pl.ds(h * n_h, n_h)].astype(jnp.bfloat16)
            part = jnp.dot(a, wb, preferred_element_type=jnp.float32)

            @pl.when(c == 0)
            def _init():
                out_ref[:, pl.ds(h * n_h, n_h)] = part

            @pl.when(jnp.logical_and(c != 0, c != N_DEV - 1))
            def _accum():
                out_ref[:, pl.ds(h * n_h, n_h)] += part

            @pl.when(c == N_DEV - 1)
            def _fin():
                s = sx_ref[0] * sw_ref[0]
                out_ref[:, pl.ds(h * n_h, n_h)] = jnp.maximum(
                    (out_ref[:, pl.ds(h * n_h, n_h)] + part) * s, 0.0
                )

        @pl.when(c < N_DEV - R)
        def _refill():
            fill_slot(r, (me + c + R) % N_DEV)

        @pl.when(c == N_DEV - 1)
        def _drain():
            for off in range(1, N_DEV):
                d = (me + off) % N_DEV
                sd = pltpu.make_async_remote_copy(
                    src_ref=x8_ref.at[pl.ds(d * m_blk, m_blk), :],
                    dst_ref=recv_ref.at[me],
                    send_sem=send_sems.at[d],
                    recv_sem=recv_sems.at[me],
                    device_id=(d,),
                    device_id_type=pl.DeviceIdType.MESH,
                )
                sd.wait_send()

    return pl.pallas_call(
        body,
        grid=(N_DEV,),
        out_shape=jax.ShapeDtypeStruct((m_blk, n_total), jnp.float32),
        in_specs=[
            pl.BlockSpec((m_total, k_shard), lambda c: (0, 0)),
            pl.BlockSpec(memory_space=pl.ANY),
            pl.BlockSpec(memory_space=pltpu.SMEM),
            pl.BlockSpec(memory_space=pltpu.SMEM),
        ],
        out_specs=pl.BlockSpec((m_blk, n_total), lambda c: (0, 0)),
        scratch_shapes=[
            pltpu.VMEM((R, m_blk, n_total), jnp.float32),
            pltpu.VMEM((m_total, k_shard), F8),
            pltpu.VMEM((N_DEV, m_blk, k_shard), F8),
            pltpu.SemaphoreType.DMA((R, S)),
            pltpu.SemaphoreType.DMA((N_DEV,)),
            pltpu.SemaphoreType.DMA((N_DEV,)),
        ],
        compiler_params=pltpu.CompilerParams(
            dimension_semantics=("arbitrary",),
            collective_id=0,
            has_side_effects=True,
            vmem_limit_bytes=64 * 1024 * 1024,
        ),
    )(x, w_mat, scale_x, scale_w)
